# Initial kernel scaffold; baseline (speedup 1.0000x reference)
import jax
import jax.numpy as jnp
from jax.experimental import pallas as pl


def kernel(x, embed_weight):
    raise NotImplementedError("write your pallas kernel here")


# speedup vs baseline: 8.2810x; 8.2810x over previous
import functools

import jax
import jax.numpy as jnp
from jax import lax
from jax.experimental import pallas as pl
from jax.experimental.pallas import tpu as pltpu
from jax.experimental.pallas import tpu_sc as plsc

N_EMBED = 8192
D_EMBED = 32
N_POINTS = 8192
BP = 1024
BN = 1024
GRID = N_POINTS // BP
JSTEPS = N_EMBED // BN
NW = 32
B_PER_W = N_POINTS // NW


def _argmin_body(xsq_ref, esq_ref, xt_ref, e_ref, idx_ref, dsum_ref):
    xt = xt_ref[...].astype(jnp.bfloat16)
    xsq = xsq_ref[0]

    def c_block(j):
        e_j = e_ref[pl.ds(j * BN, BN), :]
        esq_j = esq_ref[pl.ds(j * BN, BN), 0:1]
        mm = lax.dot_general(e_j.astype(jnp.bfloat16), xt,
                             (((1,), (0,)), ((), ())),
                             preferred_element_type=jnp.float32)
        return esq_j - 2.0 * mm

    def pass1(j, cmin):
        c = c_block(j)
        return jnp.minimum(cmin, jnp.min(c, axis=0, keepdims=True))

    cmin = lax.fori_loop(0, JSTEPS, pass1,
                         jnp.full((1, BP), jnp.inf, jnp.float32))
    m_d = xsq + cmin
    u = lax.bitcast_convert_type(m_d, jnp.int32)
    m_next = lax.bitcast_convert_type(u + 1, jnp.float32)
    h = 0.5 * (m_next - m_d)
    t1 = m_d - xsq

    def pass2(j, runidx):
        c = c_block(j)
        rows = lax.broadcasted_iota(jnp.int32, c.shape, 0) + j * BN
        cand = jnp.where((c - t1) < h, rows, jnp.int32(2 ** 30))
        return jnp.minimum(runidx, jnp.min(cand, axis=0, keepdims=True))

    runidx = lax.fori_loop(0, JSTEPS, pass2,
                           jnp.full((1, BP), 2 ** 30, jnp.int32))
    idx_ref[0] = runidx
    dsum_ref[0] = jnp.sum(m_d, keepdims=True)


_argmin_call = pl.pallas_call(
    _argmin_body,
    grid=(GRID,),
    in_specs=[
        pl.BlockSpec((1, 1, BP), lambda i: (i, 0, 0)),
        pl.BlockSpec((N_EMBED, 8), lambda i: (0, 0)),
        pl.BlockSpec((D_EMBED, BP), lambda i: (0, i)),
        pl.BlockSpec((N_EMBED, D_EMBED), lambda i: (0, 0)),
    ],
    out_specs=[
        pl.BlockSpec((1, 1, BP), lambda i: (i, 0, 0)),
        pl.BlockSpec((1, 1, 1), lambda i: (i, 0, 0)),
    ],
    out_shape=[
        jax.ShapeDtypeStruct((GRID, 1, BP), jnp.int32),
        jax.ShapeDtypeStruct((GRID, 1, 1), jnp.float32),
    ],
)


@functools.lru_cache(maxsize=1)
def _make_gather():
    mesh = plsc.VectorSubcoreMesh(core_axis_name="c", subcore_axis_name="s")

    @functools.partial(
        pl.kernel,
        mesh=mesh,
        out_type=jax.ShapeDtypeStruct((N_POINTS, D_EMBED), jnp.float32),
        scratch_types=[
            pltpu.VMEM((B_PER_W,), jnp.int32),
            pltpu.VMEM((B_PER_W, D_EMBED), jnp.float32),
            pltpu.SemaphoreType.DMA,
        ],
        compiler_params=pltpu.CompilerParams(use_tc_tiling_on_sc=False),
    )
    def _gather(table_hbm, idx_hbm, out_hbm, idx_v, rows_v, sem):
        wid = lax.axis_index("s") * 2 + lax.axis_index("c")
        base = wid * B_PER_W
        pltpu.sync_copy(idx_hbm.at[pl.ds(base, B_PER_W)], idx_v)
        pltpu.async_copy(table_hbm.at[idx_v], rows_v, sem).wait()
        pltpu.sync_copy(rows_v, out_hbm.at[pl.ds(base, B_PER_W)])

    return _gather


def kernel(x, embed_weight):
    xp_flat = jnp.transpose(x, (0, 2, 3, 1)).reshape(N_POINTS, D_EMBED)
    xp_flat = lax.optimization_barrier(xp_flat)
    xsq = jnp.sum(xp_flat ** 2, axis=1)
    esq = jnp.sum(embed_weight ** 2, axis=1)
    x_t = xp_flat.T
    idx3, dsum = _argmin_call(
        xsq.reshape(GRID, 1, BP),
        jnp.broadcast_to(esq[:, None], (N_EMBED, 8)),
        x_t, embed_weight)
    idx = idx3.reshape(N_POINTS)
    q_flat = _make_gather()(embed_weight, idx)
    q_flat = q_flat.astype(jnp.bfloat16).astype(jnp.float32)
    quantized = q_flat.reshape(8, 32, 32, D_EMBED)
    out = jnp.transpose(quantized, (0, 3, 1, 2))
    loss = jnp.sum(dsum) * jnp.float32(1.25 / 262144.0)
    return out, loss
